# Initial kernel scaffold; baseline (speedup 1.0000x reference)
import jax
import jax.numpy as jnp
from jax.experimental import pallas as pl


def kernel(x, edge_index, edge_attr, W1, root1, bias1, W2, root2, bias2):
    raise NotImplementedError("write your pallas kernel here")


# speedup vs baseline: 10.6885x; 10.6885x over previous
import functools
import jax
import jax.numpy as jnp
from jax import lax
from jax.experimental import pallas as pl
from jax.experimental.pallas import tpu as pltpu
from jax.experimental.pallas import tpu_sc as plsc

N = 10000
E = 320000
D_IN = 128
H = 128
C = 40
K = 5

NC = 2
NS = 16
NW = NC * NS
BK = 80
NR = N // 2
ZB = 16
CR = 80


def _edge_prep(src2, attr2):
    def body(s_ref, a_ref, i8_ref, i4_ref, f_ref):
        v = a_ref[...] * jnp.float32(K - 1)
        lo = jnp.floor(v)
        i0 = jnp.clip(lo.astype(jnp.int32), 0, K - 2)
        f_ref[...] = v - lo
        i8_ref[...] = s_ref[...] * 8 + i0
        i4_ref[...] = s_ref[...] * 4 + i0
    return pl.pallas_call(
        body,
        out_shape=[jax.ShapeDtypeStruct(src2.shape, jnp.int32),
                   jax.ShapeDtypeStruct(src2.shape, jnp.int32),
                   jax.ShapeDtypeStruct(src2.shape, jnp.float32)],
    )(src2, attr2)


def _matmul_bias(x, W, brow):
    M, Kd = x.shape
    F = W.shape[1]
    BM = 1000
    def body(x_ref, w_ref, b_ref, o_ref):
        o_ref[...] = lax.dot_general(
            x_ref[...], w_ref[...], (((1,), (0,)), ((), ())),
            precision=lax.Precision.HIGHEST,
            preferred_element_type=jnp.float32) + b_ref[...]
    return pl.pallas_call(
        body,
        grid=(M // BM,),
        in_specs=[pl.BlockSpec((BM, Kd), lambda i: (i, 0)),
                  pl.BlockSpec((Kd, F), lambda i: (0, 0)),
                  pl.BlockSpec((1, F), lambda i: (0, 0))],
        out_specs=pl.BlockSpec((BM, F), lambda i: (i, 0)),
        out_shape=jax.ShapeDtypeStruct((M, F), jnp.float32),
    )(x, W, brow)


def _layer_mid(agg, c0, r1, W, brow):
    F = W.shape[1]
    BM = 1000
    def body(a_ref, c0_ref, r1_ref, w_ref, b_ref, t_ref, c_ref):
        rc = 1.0 / jnp.maximum(c0_ref[...], 1.0)
        h = jnp.maximum(a_ref[...] * rc + r1_ref[...], 0.0)
        t_ref[...] = lax.dot_general(
            h, w_ref[...], (((1,), (0,)), ((), ())),
            precision=lax.Precision.HIGHEST,
            preferred_element_type=jnp.float32) + b_ref[...]
        c_ref[...] = jnp.broadcast_to(rc, (BM, H))
    return pl.pallas_call(
        body,
        grid=(N // BM,),
        in_specs=[pl.BlockSpec((BM, H), lambda i: (i, 0)),
                  pl.BlockSpec((BM, 1), lambda i: (i, 0)),
                  pl.BlockSpec((BM, H), lambda i: (i, 0)),
                  pl.BlockSpec((H, F), lambda i: (0, 0)),
                  pl.BlockSpec((1, F), lambda i: (0, 0))],
        out_specs=[pl.BlockSpec((BM, F), lambda i: (i, 0)),
                   pl.BlockSpec((BM, H), lambda i: (i, 0))],
        out_shape=[jax.ShapeDtypeStruct((N, F), jnp.float32),
                   jax.ShapeDtypeStruct((N, H), jnp.float32)],
    )(agg, c0, r1, W, brow)


def _finalize(a0, a1, rcnt, r2):
    BM = 1000
    W64 = 64
    def body(a0_ref, a1_ref, c_ref, r2_ref, o_ref):
        z = (a0_ref[...] + a1_ref[...]) * c_ref[:, 0:1] + r2_ref[...]
        col = lax.broadcasted_iota(jnp.int32, (BM, W64), 1)
        valid = col < C
        m = jnp.max(jnp.where(valid, z, -jnp.inf), axis=1, keepdims=True)
        e = jnp.where(valid, jnp.exp(z - m), 0.0)
        s = jnp.sum(e, axis=1, keepdims=True)
        o_ref[...] = z - m - jnp.log(s)
    return pl.pallas_call(
        body,
        grid=(N // BM,),
        in_specs=[pl.BlockSpec((BM, W64), lambda i: (i, 0)),
                  pl.BlockSpec((BM, W64), lambda i: (i, 0)),
                  pl.BlockSpec((BM, H), lambda i: (i, 0)),
                  pl.BlockSpec((BM, W64), lambda i: (i, 0))],
        out_specs=pl.BlockSpec((BM, W64), lambda i: (i, 0)),
        out_shape=jax.ShapeDtypeStruct((N, W64), jnp.float32),
    )(a0, a1, rcnt, r2)


WB = 125
PH = 5024
PR = PH // 2
AR = PR + 8
NZ = PR // ZB


def _edge_pass(tab, idx3, dst3, frac2, feature_split):
    nwin = idx3.shape[1]
    arows = AR + CR if feature_split else AR
    orows = 2 * PR + (CR if feature_split else 0)
    out_types = [jax.ShapeDtypeStruct((NC, orows, 128), jnp.float32)]
    scratch = [
        pltpu.VMEM((WB, BK), jnp.int32),
        pltpu.VMEM((WB, BK), jnp.int32),
        pltpu.VMEM((WB * BK,), jnp.float32),
        pltpu.VMEM((BK,), jnp.int32),
        pltpu.VMEM((BK, 128), jnp.float32),
        pltpu.VMEM((BK, 128), jnp.float32),
        pltpu.VMEM((ZB, 128), jnp.float32),
        pltpu.VMEM_SHARED((arows, 128), jnp.float32),
        pltpu.SemaphoreType.DMA,
    ]
    if feature_split:
        scratch += [
            pltpu.VMEM((CR, 128), jnp.float32),
            pltpu.VMEM((CR,), jnp.int32),
        ]
    mesh = plsc.VectorSubcoreMesh(core_axis_name="c", subcore_axis_name="s")

    @functools.partial(pl.kernel, out_type=out_types, mesh=mesh,
                       scratch_types=scratch)
    def kern(tab_h, idx_h, dst_h, frac_h, *refs):
        if feature_split:
            (out_h, idx_v, dst_v, frac_v, drow_v, rows_v, m_v,
             zero_v, agg_sh, sem, cnt_v, ident_v) = refs
        else:
            (out_h, idx_v, dst_v, frac_v, drow_v, rows_v, m_v,
             zero_v, agg_sh, sem) = refs
        cid = lax.axis_index("c")
        sid = lax.axis_index("s")
        gid = sid if feature_split else sid * NC + cid

        lane = lax.iota(jnp.int32, 16)
        zv = jnp.zeros((16,), jnp.float32)

        @pl.loop(0, ZB)
        def _zfill(i):
            for c_ in range(8):
                zero_v[i, pl.ds(c_ * 16, 16)] = zv

        if feature_split:
            @pl.loop(0, CR)
            def _cfill(i):
                for c_ in range(8):
                    cnt_v[i, pl.ds(c_ * 16, 16)] = zv
            for j in range(CR // 16):
                ident_v[pl.ds(j * 16, 16)] = lane + (AR + j * 16)
            @pl.when(sid == 0)
            def _czero():
                for j in range(CR // ZB):
                    pltpu.sync_copy(zero_v, agg_sh.at[pl.ds(AR + j * ZB, ZB)])

        nzch = (NZ - 1 - sid) // NS + 1

        def zero_agg():
            @pl.loop(0, nzch)
            def _zinit(j):
                pltpu.sync_copy(zero_v,
                                agg_sh.at[pl.ds((sid + j * NS) * ZB, ZB)])

        def edge_sweep(phase, w):
            pltpu.sync_copy(idx_h.at[gid, w], idx_v)
            pltpu.sync_copy(dst_h.at[gid, w], dst_v)
            pltpu.sync_copy(frac_h.at[gid, w], frac_v)
            if feature_split:
                off4 = cid * 4
                @pl.loop(0, WB)
                def _ixform(t):
                    for j in range(BK // 16):
                        idx_v[t, pl.ds(j * 16, 16)] = (
                            idx_v[t, pl.ds(j * 16, 16)] + off4)
            count = feature_split and phase == 0
            @pl.loop(0, WB)
            def _blk(t):
                pltpu.async_copy(tab_h.at[idx_v.at[t]], rows_v, sem).wait()
                @pl.loop(0, BK // 16)
                def _grp(g):
                    fv = frac_v[pl.ds(t * BK + g * 16, 16)]
                    dv = dst_v[t, pl.ds(g * 16, 16)]
                    dp = dv - phase * PH
                    inr = (dp >= 0) & (dp < PH)
                    drow_v[pl.ds(g * 16, 16)] = jnp.where(inr, dp >> 1, PR)
                    for j in range(16):
                        b1 = jnp.full((16,), fv[j], jnp.float32)
                        b0 = 1.0 - b1
                        d = dv[j]
                        odd = (d & 1) == 1
                        i = g * 16 + j
                        for c_ in range(4):
                            g0 = rows_v[i, pl.ds(c_ * 16, 16)]
                            g1 = rows_v[i, pl.ds(64 + c_ * 16, 16)]
                            val = g0 * b0 + g1 * b1
                            m_v[i, pl.ds(c_ * 16, 16)] = jnp.where(
                                odd, zv, val)
                            m_v[i, pl.ds(64 + c_ * 16, 16)] = jnp.where(
                                odd, val, zv)
                        if count:
                            r = d >> 7
                            cb = d & 112
                            oh = jnp.where(lane == (d & 15),
                                           jnp.float32(1.0), jnp.float32(0.0))
                            cnt_v[r, pl.ds(cb, 16)] = (
                                cnt_v[r, pl.ds(cb, 16)] + oh)
                pltpu.sync_copy(m_v, agg_sh.at[drow_v], add=True)

        for phase in range(2):
            zero_agg()
            plsc.subcore_barrier()
            for w in range(nwin):
                edge_sweep(phase, w)
            if feature_split and phase == 0:
                pltpu.sync_copy(cnt_v, agg_sh.at[ident_v], add=True)
            plsc.subcore_barrier()
            @pl.loop(0, nzch)
            def _drain(j):
                off = (sid + j * NS) * ZB
                pltpu.sync_copy(agg_sh.at[pl.ds(off, ZB)],
                                out_h.at[cid, pl.ds(phase * PR + off, ZB)])
            if phase == 0:
                plsc.subcore_barrier()
        if feature_split:
            @pl.when(sid == 0)
            def _cdrain():
                pltpu.sync_copy(agg_sh.at[pl.ds(AR, CR)],
                                out_h.at[cid, pl.ds(2 * PR, CR)])

    return kern(tab, idx3, dst3, frac2)


def kernel(x, edge_index, edge_attr, W1, root1, bias1, W2, root2, bias2):
    f32 = jnp.float32
    blocks1 = []
    for c in range(2):
        for k in range(K - 1):
            blocks1.append(W1[k][:, c * 64:(c + 1) * 64])
            blocks1.append(W1[k + 1][:, c * 64:(c + 1) * 64])
    Wbig1 = jnp.concatenate(blocks1 + [root1], axis=1)
    b1row = jnp.zeros((1, 1152), f32).at[0, 1024:].set(bias1)

    W2pad = jnp.pad(W2, ((0, 0), (0, 0), (0, 64 - C)))
    blocks2 = []
    for k in range(K - 1):
        blocks2.append(W2pad[k])
        blocks2.append(W2pad[k + 1])
    Wbig2 = jnp.concatenate(
        blocks2 + [root2, jnp.zeros((H, 64 - C), f32)], axis=1)
    b2row = jnp.zeros((1, 576), f32).at[0, 512:512 + C].set(bias2)

    src = edge_index[0]
    dst = edge_index[1]

    idx8, idx4, frac = _edge_prep(src.reshape(2500, 128),
                                  edge_attr[:, 0].reshape(2500, 128))
    epg = E // NS
    idx8g = idx8.reshape(NS, 2, WB, BK)
    dst8g = dst.reshape(NS, 2, WB, BK)
    frac8g = frac.reshape(NS, 2, WB * BK)
    epw = E // NW
    idx4w = idx4.reshape(NW, 1, WB, BK)
    dst4w = dst.reshape(NW, 1, WB, BK)
    frac4w = frac.reshape(NW, 1, WB * BK)

    outA = _matmul_bias(x, Wbig1, b1row)
    tab1 = outA[:, :1024].reshape(N * 8, 128)
    r1 = outA[:, 1024:]
    (agg1p,) = _edge_pass(tab1, idx8g, dst8g, frac8g, feature_split=True)
    agg1 = jnp.concatenate(
        [agg1p[0][:2 * PR].reshape(2 * PH, 64)[:N],
         agg1p[1][:2 * PR].reshape(2 * PH, 64)[:N]], axis=1)
    c0 = agg1p[0][2 * PR:].reshape(CR * 128, 1)[:N]

    t2r2, rcnt = _layer_mid(agg1, c0, r1, Wbig2, b2row)
    tab2 = t2r2[:, :512].reshape(N * 4, 128)
    r2 = t2r2[:, 512:]

    (agg2p,) = _edge_pass(tab2, idx4w, dst4w, frac4w, feature_split=False)

    out64 = _finalize(agg2p[0][:2 * PR].reshape(2 * PH, 64)[:N],
                      agg2p[1][:2 * PR].reshape(2 * PH, 64)[:N], rcnt, r2)
    return out64[:, :C]


# speedup vs baseline: 16.4662x; 1.5406x over previous
import functools
import jax
import jax.numpy as jnp
from jax import lax
from jax.experimental import pallas as pl
from jax.experimental.pallas import tpu as pltpu
from jax.experimental.pallas import tpu_sc as plsc

N = 10000
E = 320000
D_IN = 128
H = 128
C = 40
K = 5

NC = 2
NS = 16
NW = NC * NS
BK = 80
NR = N // 2
ZB = 16
CR = 80


def _edge_prep(src2, attr2):
    def body(s_ref, a_ref, i8_ref, i4_ref, f_ref):
        v = a_ref[...] * jnp.float32(K - 1)
        lo = jnp.floor(v)
        i0 = jnp.clip(lo.astype(jnp.int32), 0, K - 2)
        f_ref[...] = v - lo
        i8_ref[...] = s_ref[...] * 8 + i0
        i4_ref[...] = s_ref[...] * 4 + i0
    return pl.pallas_call(
        body,
        out_shape=[jax.ShapeDtypeStruct(src2.shape, jnp.int32),
                   jax.ShapeDtypeStruct(src2.shape, jnp.int32),
                   jax.ShapeDtypeStruct(src2.shape, jnp.float32)],
    )(src2, attr2)


def _matmul_bias(x, W, brow):
    M, Kd = x.shape
    F = W.shape[1]
    BM = 1000
    def body(x_ref, w_ref, b_ref, o_ref):
        o_ref[...] = lax.dot_general(
            x_ref[...], w_ref[...], (((1,), (0,)), ((), ())),
            precision=lax.Precision.HIGHEST,
            preferred_element_type=jnp.float32) + b_ref[...]
    return pl.pallas_call(
        body,
        grid=(M // BM,),
        in_specs=[pl.BlockSpec((BM, Kd), lambda i: (i, 0)),
                  pl.BlockSpec((Kd, F), lambda i: (0, 0)),
                  pl.BlockSpec((1, F), lambda i: (0, 0))],
        out_specs=pl.BlockSpec((BM, F), lambda i: (i, 0)),
        out_shape=jax.ShapeDtypeStruct((M, F), jnp.float32),
    )(x, W, brow)


def _layer_mid(agg, c0, r1, W, brow):
    F = W.shape[1]
    BM = 1000
    def body(a_ref, c0_ref, r1_ref, w_ref, b_ref, t_ref, c_ref):
        rc = 1.0 / jnp.maximum(c0_ref[...], 1.0)
        h = jnp.maximum(a_ref[...] * rc + r1_ref[...], 0.0)
        t_ref[...] = lax.dot_general(
            h, w_ref[...], (((1,), (0,)), ((), ())),
            precision=lax.Precision.HIGHEST,
            preferred_element_type=jnp.float32) + b_ref[...]
        c_ref[...] = jnp.broadcast_to(rc, (BM, H))
    return pl.pallas_call(
        body,
        grid=(N // BM,),
        in_specs=[pl.BlockSpec((BM, H), lambda i: (i, 0)),
                  pl.BlockSpec((BM, 1), lambda i: (i, 0)),
                  pl.BlockSpec((BM, H), lambda i: (i, 0)),
                  pl.BlockSpec((H, F), lambda i: (0, 0)),
                  pl.BlockSpec((1, F), lambda i: (0, 0))],
        out_specs=[pl.BlockSpec((BM, F), lambda i: (i, 0)),
                   pl.BlockSpec((BM, H), lambda i: (i, 0))],
        out_shape=[jax.ShapeDtypeStruct((N, F), jnp.float32),
                   jax.ShapeDtypeStruct((N, H), jnp.float32)],
    )(agg, c0, r1, W, brow)


def _finalize(a0, a1, rcnt, r2):
    BM = 1000
    W64 = 64
    def body(a0_ref, a1_ref, c_ref, r2_ref, o_ref):
        z = (a0_ref[...] + a1_ref[...]) * c_ref[:, 0:1] + r2_ref[...]
        col = lax.broadcasted_iota(jnp.int32, (BM, W64), 1)
        valid = col < C
        m = jnp.max(jnp.where(valid, z, -jnp.inf), axis=1, keepdims=True)
        e = jnp.where(valid, jnp.exp(z - m), 0.0)
        s = jnp.sum(e, axis=1, keepdims=True)
        o_ref[...] = z - m - jnp.log(s)
    return pl.pallas_call(
        body,
        grid=(N // BM,),
        in_specs=[pl.BlockSpec((BM, W64), lambda i: (i, 0)),
                  pl.BlockSpec((BM, W64), lambda i: (i, 0)),
                  pl.BlockSpec((BM, H), lambda i: (i, 0)),
                  pl.BlockSpec((BM, W64), lambda i: (i, 0))],
        out_specs=pl.BlockSpec((BM, W64), lambda i: (i, 0)),
        out_shape=jax.ShapeDtypeStruct((N, W64), jnp.float32),
    )(a0, a1, rcnt, r2)


WB = 25
MR = 5008
NZ = MR // ZB


def _edge_pass(tab, idx3, dst3, frac2, feature_split):
    nwin = idx3.shape[1]
    arows = MR + CR if feature_split else MR
    out_types = [jax.ShapeDtypeStruct((NC, arows, 128), jnp.float32)]
    scratch = [
        pltpu.VMEM((WB, BK), jnp.int32),
        pltpu.VMEM((WB, BK), jnp.int32),
        pltpu.VMEM((WB, BK), jnp.float32),
        pltpu.VMEM((BK,), jnp.int32),
        pltpu.VMEM((BK, 128), jnp.float32),
        pltpu.VMEM((BK, 128), jnp.float32),
        pltpu.VMEM((ZB, 128), jnp.float32),
        pltpu.VMEM_SHARED((arows, 128), jnp.float32),
        pltpu.SemaphoreType.DMA,
    ]
    if feature_split:
        scratch += [
            pltpu.VMEM((CR, 128), jnp.float32),
            pltpu.VMEM((CR,), jnp.int32),
        ]
    mesh = plsc.VectorSubcoreMesh(core_axis_name="c", subcore_axis_name="s")

    @functools.partial(pl.kernel, out_type=out_types, mesh=mesh,
                       scratch_types=scratch)
    def kern(tab_h, idx_h, dst_h, frac_h, *refs):
        if feature_split:
            (out_h, idx_v, dst_v, frac_v, drow_v, rows_v, m_v,
             zero_v, agg_sh, sem, cnt_v, ident_v) = refs
        else:
            (out_h, idx_v, dst_v, frac_v, drow_v, rows_v, m_v,
             zero_v, agg_sh, sem) = refs
        cid = lax.axis_index("c")
        sid = lax.axis_index("s")
        gid = sid if feature_split else sid * NC + cid

        lane = lax.iota(jnp.int32, 16)
        zv = jnp.zeros((16,), jnp.float32)

        @pl.loop(0, ZB)
        def _zfill(i):
            for c_ in range(8):
                zero_v[i, pl.ds(c_ * 16, 16)] = zv

        if feature_split:
            @pl.loop(0, CR)
            def _cfill(i):
                for c_ in range(8):
                    cnt_v[i, pl.ds(c_ * 16, 16)] = zv
            for j in range(CR // 16):
                ident_v[pl.ds(j * 16, 16)] = lane + (MR + j * 16)
            @pl.when(sid == 0)
            def _czero():
                for j in range(CR // ZB):
                    pltpu.sync_copy(zero_v, agg_sh.at[pl.ds(MR + j * ZB, ZB)])

        nzch = (NZ - 1 - sid) // NS + 1
        @pl.loop(0, nzch)
        def _zinit(j):
            pltpu.sync_copy(zero_v, agg_sh.at[pl.ds((sid + j * NS) * ZB, ZB)])
        plsc.subcore_barrier()

        for w in range(nwin):
            pltpu.sync_copy(idx_h.at[gid, w], idx_v)
            pltpu.sync_copy(dst_h.at[gid, w], dst_v)
            pltpu.sync_copy(frac_h.at[gid, w], frac_v)
            if feature_split:
                off4 = cid * 4
                @pl.loop(0, WB)
                def _ixform(t):
                    for j in range(BK // 16):
                        idx_v[t, pl.ds(j * 16, 16)] = (
                            idx_v[t, pl.ds(j * 16, 16)] + off4)
            @pl.loop(0, WB)
            def _blk(t):
                pltpu.async_copy(tab_h.at[idx_v.at[t]], rows_v, sem).wait()
                @pl.loop(0, BK // 16)
                def _grp(g):
                    fv = frac_v[t, pl.ds(g * 16, 16)]
                    dv = dst_v[t, pl.ds(g * 16, 16)]
                    drow_v[pl.ds(g * 16, 16)] = dv >> 1
                    for j in range(16):
                        b1 = jnp.full((16,), fv[j], jnp.float32)
                        b0 = 1.0 - b1
                        d = dv[j]
                        odd = (d & 1) == 1
                        i = g * 16 + j
                        for c_ in range(4):
                            g0 = rows_v[i, pl.ds(c_ * 16, 16)]
                            g1 = rows_v[i, pl.ds(64 + c_ * 16, 16)]
                            val = g0 * b0 + g1 * b1
                            m_v[i, pl.ds(c_ * 16, 16)] = jnp.where(
                                odd, zv, val)
                            m_v[i, pl.ds(64 + c_ * 16, 16)] = jnp.where(
                                odd, val, zv)
                        if feature_split:
                            r = d >> 7
                            cb = d & 112
                            oh = jnp.where(lane == (d & 15),
                                           jnp.float32(1.0), jnp.float32(0.0))
                            cnt_v[r, pl.ds(cb, 16)] = (
                                cnt_v[r, pl.ds(cb, 16)] + oh)
                pltpu.sync_copy(m_v, agg_sh.at[drow_v], add=True)

        if feature_split:
            pltpu.sync_copy(cnt_v, agg_sh.at[ident_v], add=True)
        plsc.subcore_barrier()

        @pl.loop(0, nzch)
        def _drain(j):
            off = (sid + j * NS) * ZB
            pltpu.sync_copy(agg_sh.at[pl.ds(off, ZB)],
                            out_h.at[cid, pl.ds(off, ZB)])
        if feature_split:
            @pl.when(sid == 0)
            def _cdrain():
                for j in range(CR // ZB):
                    pltpu.sync_copy(agg_sh.at[pl.ds(MR + j * ZB, ZB)],
                                    out_h.at[cid, pl.ds(MR + j * ZB, ZB)])

    return kern(tab, idx3, dst3, frac2)


def kernel(x, edge_index, edge_attr, W1, root1, bias1, W2, root2, bias2):
    f32 = jnp.float32
    blocks1 = []
    for c in range(2):
        for k in range(K - 1):
            blocks1.append(W1[k][:, c * 64:(c + 1) * 64])
            blocks1.append(W1[k + 1][:, c * 64:(c + 1) * 64])
    Wbig1 = jnp.concatenate(blocks1 + [root1], axis=1)
    b1row = jnp.zeros((1, 1152), f32).at[0, 1024:].set(bias1)

    W2pad = jnp.pad(W2, ((0, 0), (0, 0), (0, 64 - C)))
    blocks2 = []
    for k in range(K - 1):
        blocks2.append(W2pad[k])
        blocks2.append(W2pad[k + 1])
    Wbig2 = jnp.concatenate(
        blocks2 + [root2, jnp.zeros((H, 64 - C), f32)], axis=1)
    b2row = jnp.zeros((1, 576), f32).at[0, 512:512 + C].set(bias2)

    src = edge_index[0]
    dst = edge_index[1]

    idx8, idx4, frac = _edge_prep(src.reshape(2500, 128),
                                  edge_attr[:, 0].reshape(2500, 128))
    nw1 = E // NS // (WB * BK)
    idx8g = idx8.reshape(NS, nw1, WB, BK)
    dst8g = dst.reshape(NS, nw1, WB, BK)
    frac8g = frac.reshape(NS, nw1, WB, BK)
    nw2 = E // NW // (WB * BK)
    idx4w = idx4.reshape(NW, nw2, WB, BK)
    dst4w = dst.reshape(NW, nw2, WB, BK)
    frac4w = frac.reshape(NW, nw2, WB, BK)

    outA = _matmul_bias(x, Wbig1, b1row)
    tab1 = outA[:, :1024].reshape(N * 8, 128)
    r1 = outA[:, 1024:]
    (agg1p,) = _edge_pass(tab1, idx8g, dst8g, frac8g, feature_split=True)
    agg1 = jnp.concatenate(
        [agg1p[0][:MR].reshape(2 * MR, 64)[:N],
         agg1p[1][:MR].reshape(2 * MR, 64)[:N]], axis=1)
    c0 = agg1p[0][MR:].reshape(CR * 128, 1)[:N]

    t2r2, rcnt = _layer_mid(agg1, c0, r1, Wbig2, b2row)
    tab2 = t2r2[:, :512].reshape(N * 4, 128)
    r2 = t2r2[:, 512:]

    (agg2p,) = _edge_pass(tab2, idx4w, dst4w, frac4w, feature_split=False)

    out64 = _finalize(agg2p[0].reshape(2 * MR, 64)[:N],
                      agg2p[1].reshape(2 * MR, 64)[:N], rcnt, r2)
    return out64[:, :C]


# speedup vs baseline: 23.7690x; 1.4435x over previous
import functools
import jax
import jax.numpy as jnp
from jax import lax
from jax.experimental import pallas as pl
from jax.experimental.pallas import tpu as pltpu
from jax.experimental.pallas import tpu_sc as plsc

N = 10000
E = 320000
D_IN = 128
H = 128
C = 40
K = 5

NC = 2
NS = 16
NW = NC * NS
BK = 80
NR = N // 2
ZB = 16
CR = 80


def _edge_prep(src2, attr2):
    def body(s_ref, a_ref, i8_ref, i4_ref, f_ref):
        v = a_ref[...] * jnp.float32(K - 1)
        lo = jnp.floor(v)
        i0 = jnp.clip(lo.astype(jnp.int32), 0, K - 2)
        f_ref[...] = v - lo
        i8_ref[...] = s_ref[...] * 8 + i0
        i4_ref[...] = s_ref[...] * 4 + i0
    return pl.pallas_call(
        body,
        out_shape=[jax.ShapeDtypeStruct(src2.shape, jnp.int32),
                   jax.ShapeDtypeStruct(src2.shape, jnp.int32),
                   jax.ShapeDtypeStruct(src2.shape, jnp.float32)],
    )(src2, attr2)


def _matmul_bias(x, W, brow):
    M, Kd = x.shape
    F = W.shape[1]
    BM = 1000
    def body(x_ref, w_ref, b_ref, o_ref):
        o_ref[...] = lax.dot_general(
            x_ref[...], w_ref[...], (((1,), (0,)), ((), ())),
            precision=lax.Precision.HIGHEST,
            preferred_element_type=jnp.float32) + b_ref[...]
    return pl.pallas_call(
        body,
        grid=(M // BM,),
        in_specs=[pl.BlockSpec((BM, Kd), lambda i: (i, 0)),
                  pl.BlockSpec((Kd, F), lambda i: (0, 0)),
                  pl.BlockSpec((1, F), lambda i: (0, 0))],
        out_specs=pl.BlockSpec((BM, F), lambda i: (i, 0)),
        out_shape=jax.ShapeDtypeStruct((M, F), jnp.float32),
    )(x, W, brow)


def _layer_mid(agg, c0, r1, W, brow):
    F = W.shape[1]
    BM = 1000
    def body(a_ref, c0_ref, r1_ref, w_ref, b_ref, t_ref, c_ref):
        rc = 1.0 / jnp.maximum(c0_ref[...], 1.0)
        h = jnp.maximum(a_ref[...] * rc + r1_ref[...], 0.0)
        t_ref[...] = lax.dot_general(
            h, w_ref[...], (((1,), (0,)), ((), ())),
            precision=lax.Precision.HIGHEST,
            preferred_element_type=jnp.float32) + b_ref[...]
        c_ref[...] = jnp.broadcast_to(rc, (BM, H))
    return pl.pallas_call(
        body,
        grid=(N // BM,),
        in_specs=[pl.BlockSpec((BM, H), lambda i: (i, 0)),
                  pl.BlockSpec((BM, 1), lambda i: (i, 0)),
                  pl.BlockSpec((BM, H), lambda i: (i, 0)),
                  pl.BlockSpec((H, F), lambda i: (0, 0)),
                  pl.BlockSpec((1, F), lambda i: (0, 0))],
        out_specs=[pl.BlockSpec((BM, F), lambda i: (i, 0)),
                   pl.BlockSpec((BM, H), lambda i: (i, 0))],
        out_shape=[jax.ShapeDtypeStruct((N, F), jnp.float32),
                   jax.ShapeDtypeStruct((N, H), jnp.float32)],
    )(agg, c0, r1, W, brow)


def _finalize(a0, a1, rcnt, r2):
    BM = 1000
    W64 = 64
    def body(a0_ref, a1_ref, c_ref, r2_ref, o_ref):
        z = (a0_ref[...] + a1_ref[...]) * c_ref[:, 0:1] + r2_ref[...]
        col = lax.broadcasted_iota(jnp.int32, (BM, W64), 1)
        valid = col < C
        m = jnp.max(jnp.where(valid, z, -jnp.inf), axis=1, keepdims=True)
        e = jnp.where(valid, jnp.exp(z - m), 0.0)
        s = jnp.sum(e, axis=1, keepdims=True)
        o_ref[...] = z - m - jnp.log(s)
    return pl.pallas_call(
        body,
        grid=(N // BM,),
        in_specs=[pl.BlockSpec((BM, W64), lambda i: (i, 0)),
                  pl.BlockSpec((BM, W64), lambda i: (i, 0)),
                  pl.BlockSpec((BM, H), lambda i: (i, 0)),
                  pl.BlockSpec((BM, W64), lambda i: (i, 0))],
        out_specs=pl.BlockSpec((BM, W64), lambda i: (i, 0)),
        out_shape=jax.ShapeDtypeStruct((N, W64), jnp.float32),
    )(a0, a1, rcnt, r2)


WB = 25
MR = 5008
NZ = MR // ZB


def _edge_pass(tab, idx3, dst3, frac2, feature_split):
    nwin = idx3.shape[1]
    arows = MR + CR if feature_split else MR
    out_types = [jax.ShapeDtypeStruct((NC, arows, 128), jnp.float32)]
    scratch = [
        pltpu.VMEM((WB, BK), jnp.int32),
        pltpu.VMEM((WB, BK), jnp.int32),
        pltpu.VMEM((WB, BK), jnp.float32),
        pltpu.VMEM((BK,), jnp.int32),
        pltpu.VMEM((BK,), jnp.int32),
        pltpu.VMEM((BK, 128), jnp.float32),
        pltpu.VMEM((BK, 128), jnp.float32),
        pltpu.VMEM((BK, 128), jnp.float32),
        pltpu.VMEM((BK, 128), jnp.float32),
        pltpu.VMEM((ZB, 128), jnp.float32),
        pltpu.VMEM_SHARED((arows, 128), jnp.float32),
        pltpu.SemaphoreType.DMA,
        pltpu.SemaphoreType.DMA,
        pltpu.SemaphoreType.DMA,
        pltpu.SemaphoreType.DMA,
    ]
    if feature_split:
        scratch += [
            pltpu.VMEM((CR, 128), jnp.float32),
            pltpu.VMEM((CR,), jnp.int32),
        ]
    mesh = plsc.VectorSubcoreMesh(core_axis_name="c", subcore_axis_name="s")

    @functools.partial(pl.kernel, out_type=out_types, mesh=mesh,
                       scratch_types=scratch)
    def kern(tab_h, idx_h, dst_h, frac_h, *refs):
        if feature_split:
            (out_h, idx_v, dst_v, frac_v, drow_a, drow_b, rows_a, rows_b,
             m_a, m_b, zero_v, agg_sh, semga, semgb, semsa, semsb,
             cnt_v, ident_v) = refs
        else:
            (out_h, idx_v, dst_v, frac_v, drow_a, drow_b, rows_a, rows_b,
             m_a, m_b, zero_v, agg_sh, semga, semgb, semsa, semsb) = refs
        cid = lax.axis_index("c")
        sid = lax.axis_index("s")
        gid = sid if feature_split else sid * NC + cid

        lane = lax.iota(jnp.int32, 16)
        zv = jnp.zeros((16,), jnp.float32)

        @pl.loop(0, ZB)
        def _zfill(i):
            for c_ in range(8):
                zero_v[i, pl.ds(c_ * 16, 16)] = zv

        if feature_split:
            @pl.loop(0, CR)
            def _cfill(i):
                for c_ in range(8):
                    cnt_v[i, pl.ds(c_ * 16, 16)] = zv
            for j in range(CR // 16):
                ident_v[pl.ds(j * 16, 16)] = lane + (MR + j * 16)
            @pl.when(sid == 0)
            def _czero():
                for j in range(CR // ZB):
                    pltpu.sync_copy(zero_v, agg_sh.at[pl.ds(MR + j * ZB, ZB)])

        nzch = (NZ - 1 - sid) // NS + 1
        @pl.loop(0, nzch)
        def _zinit(j):
            pltpu.sync_copy(zero_v, agg_sh.at[pl.ds((sid + j * NS) * ZB, ZB)])
        plsc.subcore_barrier()

        @pl.loop(0, nwin)
        def _win(w):
            pltpu.sync_copy(idx_h.at[gid, w], idx_v)
            pltpu.sync_copy(dst_h.at[gid, w], dst_v)
            pltpu.sync_copy(frac_h.at[gid, w], frac_v)
            if feature_split:
                off4 = cid * 4
                @pl.loop(0, WB)
                def _ixform(t):
                    for j in range(BK // 16):
                        idx_v[t, pl.ds(j * 16, 16)] = (
                            idx_v[t, pl.ds(j * 16, 16)] + off4)
            def compute(t, rows_v, m_v, drow_v):
                @pl.loop(0, BK // 16)
                def _grp(g):
                    fv = frac_v[t, pl.ds(g * 16, 16)]
                    dv = dst_v[t, pl.ds(g * 16, 16)]
                    drow_v[pl.ds(g * 16, 16)] = dv >> 1
                    for j in range(16):
                        b1 = jnp.full((16,), fv[j], jnp.float32)
                        b0 = 1.0 - b1
                        d = dv[j]
                        odd = (d & 1) == 1
                        i = g * 16 + j
                        for c_ in range(4):
                            g0 = rows_v[i, pl.ds(c_ * 16, 16)]
                            g1 = rows_v[i, pl.ds(64 + c_ * 16, 16)]
                            val = g0 * b0 + g1 * b1
                            m_v[i, pl.ds(c_ * 16, 16)] = jnp.where(
                                odd, zv, val)
                            m_v[i, pl.ds(64 + c_ * 16, 16)] = jnp.where(
                                odd, val, zv)
                        if feature_split:
                            r = d >> 7
                            cb = d & 112
                            oh = jnp.where(lane == (d & 15),
                                           jnp.float32(1.0), jnp.float32(0.0))
                            cnt_v[r, pl.ds(cb, 16)] = (
                                cnt_v[r, pl.ds(cb, 16)] + oh)

            pltpu.async_copy(tab_h.at[idx_v.at[0]], rows_a, semga)
            @pl.loop(0, WB // 2)
            def _pair(tt):
                t0 = 2 * tt
                pltpu.async_copy(tab_h.at[idx_v.at[t0 + 1]], rows_b, semgb)
                pltpu.make_async_copy(tab_h.at[idx_v.at[t0]], rows_a,
                                      semga).wait()
                compute(t0, rows_a, m_a, drow_a)
                sca = pltpu.async_copy(m_a, agg_sh.at[drow_a], semsa,
                                       add=True)
                @pl.when(tt + 1 < WB // 2)
                def _pre():
                    pltpu.async_copy(tab_h.at[idx_v.at[t0 + 2]], rows_a,
                                     semga)
                pltpu.make_async_copy(tab_h.at[idx_v.at[t0 + 1]], rows_b,
                                      semgb).wait()
                compute(t0 + 1, rows_b, m_b, drow_b)
                scb = pltpu.async_copy(m_b, agg_sh.at[drow_b], semsb,
                                       add=True)
                sca.wait()
                scb.wait()
            if WB % 2 == 1:
                t_last = WB - 1
                pltpu.async_copy(tab_h.at[idx_v.at[t_last]], rows_a,
                                 semga).wait()
                compute(t_last, rows_a, m_a, drow_a)
                pltpu.sync_copy(m_a, agg_sh.at[drow_a], add=True)

        if feature_split:
            pltpu.sync_copy(cnt_v, agg_sh.at[ident_v], add=True)
        plsc.subcore_barrier()

        @pl.loop(0, nzch)
        def _drain(j):
            off = (sid + j * NS) * ZB
            pltpu.sync_copy(agg_sh.at[pl.ds(off, ZB)],
                            out_h.at[cid, pl.ds(off, ZB)])
        if feature_split:
            @pl.when(sid == 0)
            def _cdrain():
                for j in range(CR // ZB):
                    pltpu.sync_copy(agg_sh.at[pl.ds(MR + j * ZB, ZB)],
                                    out_h.at[cid, pl.ds(MR + j * ZB, ZB)])

    return kern(tab, idx3, dst3, frac2)


def kernel(x, edge_index, edge_attr, W1, root1, bias1, W2, root2, bias2):
    f32 = jnp.float32
    blocks1 = []
    for c in range(2):
        for k in range(K - 1):
            blocks1.append(W1[k][:, c * 64:(c + 1) * 64])
            blocks1.append(W1[k + 1][:, c * 64:(c + 1) * 64])
    Wbig1 = jnp.concatenate(blocks1 + [root1], axis=1)
    b1row = jnp.zeros((1, 1152), f32).at[0, 1024:].set(bias1)

    W2pad = jnp.pad(W2, ((0, 0), (0, 0), (0, 64 - C)))
    blocks2 = []
    for k in range(K - 1):
        blocks2.append(W2pad[k])
        blocks2.append(W2pad[k + 1])
    Wbig2 = jnp.concatenate(
        blocks2 + [root2, jnp.zeros((H, 64 - C), f32)], axis=1)
    b2row = jnp.zeros((1, 576), f32).at[0, 512:512 + C].set(bias2)

    src = edge_index[0]
    dst = edge_index[1]

    idx8, idx4, frac = _edge_prep(src.reshape(2500, 128),
                                  edge_attr[:, 0].reshape(2500, 128))
    nw1 = E // NS // (WB * BK)
    idx8g = idx8.reshape(NS, nw1, WB, BK)
    dst8g = dst.reshape(NS, nw1, WB, BK)
    frac8g = frac.reshape(NS, nw1, WB, BK)
    nw2 = E // NW // (WB * BK)
    idx4w = idx4.reshape(NW, nw2, WB, BK)
    dst4w = dst.reshape(NW, nw2, WB, BK)
    frac4w = frac.reshape(NW, nw2, WB, BK)

    outA = _matmul_bias(x, Wbig1, b1row)
    tab1 = outA[:, :1024].reshape(N * 8, 128)
    r1 = outA[:, 1024:]
    (agg1p,) = _edge_pass(tab1, idx8g, dst8g, frac8g, feature_split=True)
    agg1 = jnp.concatenate(
        [agg1p[0][:MR].reshape(2 * MR, 64)[:N],
         agg1p[1][:MR].reshape(2 * MR, 64)[:N]], axis=1)
    c0 = agg1p[0][MR:].reshape(CR * 128, 1)[:N]

    t2r2, rcnt = _layer_mid(agg1, c0, r1, Wbig2, b2row)
    tab2 = t2r2[:, :512].reshape(N * 4, 128)
    r2 = t2r2[:, 512:]

    (agg2p,) = _edge_pass(tab2, idx4w, dst4w, frac4w, feature_split=False)

    out64 = _finalize(agg2p[0].reshape(2 * MR, 64)[:N],
                      agg2p[1].reshape(2 * MR, 64)[:N], rcnt, r2)
    return out64[:, :C]


# speedup vs baseline: 25.1746x; 1.0591x over previous
import functools
import jax
import jax.numpy as jnp
from jax import lax
from jax.experimental import pallas as pl
from jax.experimental.pallas import tpu as pltpu
from jax.experimental.pallas import tpu_sc as plsc

N = 10000
E = 320000
D_IN = 128
H = 128
C = 40
K = 5

NC = 2
NS = 16
NW = NC * NS
BK = 80
NR = N // 2
ZB = 16
CR = 80


def _front(x, W, b1, src2, attr2):
    BM = 1000
    def body(x_ref, w_ref, b_ref, s_ref, a_ref,
             t_ref, r_ref, i8_ref, i4_ref, f_ref):
        o = lax.dot_general(
            x_ref[...], w_ref[...], (((1,), (0,)), ((), ())),
            precision=lax.Precision.HIGHEST,
            preferred_element_type=jnp.float32)
        t_ref[...] = o[:, :1024]
        r_ref[...] = o[:, 1024:] + b_ref[...]
        v = a_ref[...] * jnp.float32(K - 1)
        lo = jnp.floor(v)
        i0 = jnp.clip(lo.astype(jnp.int32), 0, K - 2)
        f_ref[...] = v - lo
        i8_ref[...] = s_ref[...] * 8 + i0
        i4_ref[...] = s_ref[...] * 4 + i0
    return pl.pallas_call(
        body,
        grid=(N // BM,),
        in_specs=[pl.BlockSpec((BM, 128), lambda i: (i, 0)),
                  pl.BlockSpec((128, 1152), lambda i: (0, 0)),
                  pl.BlockSpec((1, 128), lambda i: (0, 0)),
                  pl.BlockSpec((256, 128), lambda i: (i, 0)),
                  pl.BlockSpec((256, 128), lambda i: (i, 0))],
        out_specs=[pl.BlockSpec((BM, 1024), lambda i: (i, 0)),
                   pl.BlockSpec((BM, 128), lambda i: (i, 0)),
                   pl.BlockSpec((256, 128), lambda i: (i, 0)),
                   pl.BlockSpec((256, 128), lambda i: (i, 0)),
                   pl.BlockSpec((256, 128), lambda i: (i, 0))],
        out_shape=[jax.ShapeDtypeStruct((N, 1024), jnp.float32),
                   jax.ShapeDtypeStruct((N, 128), jnp.float32),
                   jax.ShapeDtypeStruct((2560, 128), jnp.int32),
                   jax.ShapeDtypeStruct((2560, 128), jnp.int32),
                   jax.ShapeDtypeStruct((2560, 128), jnp.float32)],
    )(x, W, b1, src2, attr2)


def _layer_mid(agg, c0, r1, W, brow):
    BM = 1000
    def body(a_ref, c0_ref, r1_ref, w_ref, b_ref, t_ref, r_ref, c_ref):
        rc = 1.0 / jnp.maximum(c0_ref[...], 1.0)
        h = jnp.maximum(a_ref[...] * rc + r1_ref[...], 0.0)
        o = lax.dot_general(
            h, w_ref[...], (((1,), (0,)), ((), ())),
            precision=lax.Precision.HIGHEST,
            preferred_element_type=jnp.float32) + b_ref[...]
        t_ref[...] = o[:, :512]
        r_ref[...] = o[:, 512:]
        c_ref[...] = jnp.broadcast_to(rc, (BM, H))
    return pl.pallas_call(
        body,
        grid=(N // BM,),
        in_specs=[pl.BlockSpec((BM, H), lambda i: (i, 0)),
                  pl.BlockSpec((BM, 1), lambda i: (i, 0)),
                  pl.BlockSpec((BM, H), lambda i: (i, 0)),
                  pl.BlockSpec((H, 576), lambda i: (0, 0)),
                  pl.BlockSpec((1, 576), lambda i: (0, 0))],
        out_specs=[pl.BlockSpec((BM, 512), lambda i: (i, 0)),
                   pl.BlockSpec((BM, 64), lambda i: (i, 0)),
                   pl.BlockSpec((BM, H), lambda i: (i, 0))],
        out_shape=[jax.ShapeDtypeStruct((N, 512), jnp.float32),
                   jax.ShapeDtypeStruct((N, 64), jnp.float32),
                   jax.ShapeDtypeStruct((N, H), jnp.float32)],
    )(agg, c0, r1, W, brow)


def _finalize(a0, a1, rcnt, r2):
    BM = 1000
    W64 = 64
    def body(a0_ref, a1_ref, c_ref, r2_ref, o_ref):
        z = (a0_ref[...] + a1_ref[...]) * c_ref[:, 0:1] + r2_ref[...]
        col = lax.broadcasted_iota(jnp.int32, (BM, W64), 1)
        valid = col < C
        m = jnp.max(jnp.where(valid, z, -jnp.inf), axis=1, keepdims=True)
        e = jnp.where(valid, jnp.exp(z - m), 0.0)
        s = jnp.sum(e, axis=1, keepdims=True)
        o_ref[...] = z - m - jnp.log(s)
    return pl.pallas_call(
        body,
        grid=(N // BM,),
        in_specs=[pl.BlockSpec((BM, W64), lambda i: (i, 0)),
                  pl.BlockSpec((BM, W64), lambda i: (i, 0)),
                  pl.BlockSpec((BM, H), lambda i: (i, 0)),
                  pl.BlockSpec((BM, W64), lambda i: (i, 0))],
        out_specs=pl.BlockSpec((BM, W64), lambda i: (i, 0)),
        out_shape=jax.ShapeDtypeStruct((N, W64), jnp.float32),
    )(a0, a1, rcnt, r2)


WB = 25
MR = 5008
NZ = MR // ZB


def _edge_pass(tab, idx3, dst3, frac2, feature_split):
    nwin = idx3.shape[1]
    arows = MR + CR if feature_split else MR
    out_types = [jax.ShapeDtypeStruct((NC, arows, 128), jnp.float32)]
    scratch = [
        pltpu.VMEM((WB, BK), jnp.int32),
        pltpu.VMEM((WB, BK), jnp.int32),
        pltpu.VMEM((WB, BK), jnp.float32),
        pltpu.VMEM((BK,), jnp.int32),
        pltpu.VMEM((BK,), jnp.int32),
        pltpu.VMEM((BK, 128), jnp.float32),
        pltpu.VMEM((BK, 128), jnp.float32),
        pltpu.VMEM((BK, 128), jnp.float32),
        pltpu.VMEM((BK, 128), jnp.float32),
        pltpu.VMEM((ZB, 128), jnp.float32),
        pltpu.VMEM_SHARED((arows, 128), jnp.float32),
        pltpu.SemaphoreType.DMA,
        pltpu.SemaphoreType.DMA,
        pltpu.SemaphoreType.DMA,
        pltpu.SemaphoreType.DMA,
    ]
    if feature_split:
        scratch += [
            pltpu.VMEM((CR, 128), jnp.float32),
            pltpu.VMEM((CR,), jnp.int32),
        ]
    mesh = plsc.VectorSubcoreMesh(core_axis_name="c", subcore_axis_name="s")

    @functools.partial(pl.kernel, out_type=out_types, mesh=mesh,
                       scratch_types=scratch)
    def kern(tab_h, idx_h, dst_h, frac_h, *refs):
        if feature_split:
            (out_h, idx_v, dst_v, frac_v, drow_a, drow_b, rows_a, rows_b,
             m_a, m_b, zero_v, agg_sh, semga, semgb, semsa, semsb,
             cnt_v, ident_v) = refs
        else:
            (out_h, idx_v, dst_v, frac_v, drow_a, drow_b, rows_a, rows_b,
             m_a, m_b, zero_v, agg_sh, semga, semgb, semsa, semsb) = refs
        cid = lax.axis_index("c")
        sid = lax.axis_index("s")
        gid = sid if feature_split else sid * NC + cid

        lane = lax.iota(jnp.int32, 16)
        zv = jnp.zeros((16,), jnp.float32)

        @pl.loop(0, ZB)
        def _zfill(i):
            for c_ in range(8):
                zero_v[i, pl.ds(c_ * 16, 16)] = zv

        if feature_split:
            @pl.loop(0, CR)
            def _cfill(i):
                for c_ in range(8):
                    cnt_v[i, pl.ds(c_ * 16, 16)] = zv
            for j in range(CR // 16):
                ident_v[pl.ds(j * 16, 16)] = lane + (MR + j * 16)
            @pl.when(sid == 0)
            def _czero():
                for j in range(CR // ZB):
                    pltpu.sync_copy(zero_v, agg_sh.at[pl.ds(MR + j * ZB, ZB)])

        nzch = (NZ - 1 - sid) // NS + 1
        @pl.loop(0, nzch)
        def _zinit(j):
            pltpu.sync_copy(zero_v, agg_sh.at[pl.ds((sid + j * NS) * ZB, ZB)])
        plsc.subcore_barrier()

        @pl.loop(0, nwin)
        def _win(w):
            pltpu.sync_copy(idx_h.at[gid, w], idx_v)
            pltpu.sync_copy(dst_h.at[gid, w], dst_v)
            pltpu.sync_copy(frac_h.at[gid, w], frac_v)
            if feature_split:
                off4 = cid * 4
                @pl.loop(0, WB)
                def _ixform(t):
                    for j in range(BK // 16):
                        idx_v[t, pl.ds(j * 16, 16)] = (
                            idx_v[t, pl.ds(j * 16, 16)] + off4)
            def compute(t, rows_v, m_v, drow_v):
                @pl.loop(0, BK // 16)
                def _grp(g):
                    fv = frac_v[t, pl.ds(g * 16, 16)]
                    dv = dst_v[t, pl.ds(g * 16, 16)]
                    drow_v[pl.ds(g * 16, 16)] = dv >> 1
                    for j in range(16):
                        b1 = jnp.full((16,), fv[j], jnp.float32)
                        b0 = 1.0 - b1
                        d = dv[j]
                        odd = (d & 1) == 1
                        i = g * 16 + j
                        for c_ in range(4):
                            g0 = rows_v[i, pl.ds(c_ * 16, 16)]
                            g1 = rows_v[i, pl.ds(64 + c_ * 16, 16)]
                            val = g0 * b0 + g1 * b1
                            m_v[i, pl.ds(c_ * 16, 16)] = jnp.where(
                                odd, zv, val)
                            m_v[i, pl.ds(64 + c_ * 16, 16)] = jnp.where(
                                odd, val, zv)
                        if feature_split:
                            r = d >> 7
                            cb = d & 112
                            oh = jnp.where(lane == (d & 15),
                                           jnp.float32(1.0), jnp.float32(0.0))
                            cnt_v[r, pl.ds(cb, 16)] = (
                                cnt_v[r, pl.ds(cb, 16)] + oh)

            pltpu.async_copy(tab_h.at[idx_v.at[0]], rows_a, semga)
            @pl.loop(0, WB // 2)
            def _pair(tt):
                t0 = 2 * tt
                pltpu.async_copy(tab_h.at[idx_v.at[t0 + 1]], rows_b, semgb)
                pltpu.make_async_copy(tab_h.at[idx_v.at[t0]], rows_a,
                                      semga).wait()
                compute(t0, rows_a, m_a, drow_a)
                sca = pltpu.async_copy(m_a, agg_sh.at[drow_a], semsa,
                                       add=True)
                @pl.when(tt + 1 < WB // 2)
                def _pre():
                    pltpu.async_copy(tab_h.at[idx_v.at[t0 + 2]], rows_a,
                                     semga)
                pltpu.make_async_copy(tab_h.at[idx_v.at[t0 + 1]], rows_b,
                                      semgb).wait()
                compute(t0 + 1, rows_b, m_b, drow_b)
                scb = pltpu.async_copy(m_b, agg_sh.at[drow_b], semsb,
                                       add=True)
                sca.wait()
                scb.wait()
            if WB % 2 == 1:
                t_last = WB - 1
                pltpu.async_copy(tab_h.at[idx_v.at[t_last]], rows_a,
                                 semga).wait()
                compute(t_last, rows_a, m_a, drow_a)
                pltpu.sync_copy(m_a, agg_sh.at[drow_a], add=True)

        if feature_split:
            pltpu.sync_copy(cnt_v, agg_sh.at[ident_v], add=True)
        plsc.subcore_barrier()

        @pl.loop(0, nzch)
        def _drain(j):
            off = (sid + j * NS) * ZB
            pltpu.sync_copy(agg_sh.at[pl.ds(off, ZB)],
                            out_h.at[cid, pl.ds(off, ZB)])
        if feature_split:
            @pl.when(sid == 0)
            def _cdrain():
                for j in range(CR // ZB):
                    pltpu.sync_copy(agg_sh.at[pl.ds(MR + j * ZB, ZB)],
                                    out_h.at[cid, pl.ds(MR + j * ZB, ZB)])

    return kern(tab, idx3, dst3, frac2)


def kernel(x, edge_index, edge_attr, W1, root1, bias1, W2, root2, bias2):
    f32 = jnp.float32
    blocks1 = []
    for c in range(2):
        for k in range(K - 1):
            blocks1.append(W1[k][:, c * 64:(c + 1) * 64])
            blocks1.append(W1[k + 1][:, c * 64:(c + 1) * 64])
    Wbig1 = jnp.concatenate(blocks1 + [root1], axis=1)
    b1row = bias1.reshape(1, 128)

    W2pad = jnp.pad(W2, ((0, 0), (0, 0), (0, 64 - C)))
    blocks2 = []
    for k in range(K - 1):
        blocks2.append(W2pad[k])
        blocks2.append(W2pad[k + 1])
    Wbig2 = jnp.concatenate(
        blocks2 + [root2, jnp.zeros((H, 64 - C), f32)], axis=1)
    b2row = jnp.zeros((1, 576), f32).at[0, 512:512 + C].set(bias2)

    src = edge_index[0]
    dst = edge_index[1]

    pad60 = ((0, 60), (0, 0))
    tabA, r1, idx8p, idx4p, fracp = _front(
        x, Wbig1, b1row, jnp.pad(src.reshape(2500, 128), pad60),
        jnp.pad(edge_attr[:, 0].reshape(2500, 128), pad60))
    tab1 = tabA.reshape(N * 8, 128)
    idx8, idx4, frac = idx8p[:2500], idx4p[:2500], fracp[:2500]
    nw1 = E // NS // (WB * BK)
    idx8g = idx8.reshape(NS, nw1, WB, BK)
    dst8g = dst.reshape(NS, nw1, WB, BK)
    frac8g = frac.reshape(NS, nw1, WB, BK)
    nw2 = E // NW // (WB * BK)
    idx4w = idx4.reshape(NW, nw2, WB, BK)
    dst4w = dst.reshape(NW, nw2, WB, BK)
    frac4w = frac.reshape(NW, nw2, WB, BK)
    (agg1p,) = _edge_pass(tab1, idx8g, dst8g, frac8g, feature_split=True)
    agg1 = jnp.concatenate(
        [agg1p[0][:MR].reshape(2 * MR, 64)[:N],
         agg1p[1][:MR].reshape(2 * MR, 64)[:N]], axis=1)
    c0 = agg1p[0][MR:].reshape(CR * 128, 1)[:N]

    t2, r2, rcnt = _layer_mid(agg1, c0, r1, Wbig2, b2row)
    tab2 = t2.reshape(N * 4, 128)

    (agg2p,) = _edge_pass(tab2, idx4w, dst4w, frac4w, feature_split=False)

    out64 = _finalize(agg2p[0].reshape(2 * MR, 64)[:N],
                      agg2p[1].reshape(2 * MR, 64)[:N], rcnt, r2)
    return out64[:, :C]


# speedup vs baseline: 27.1077x; 1.0768x over previous
import functools
import jax
import jax.numpy as jnp
from jax import lax
from jax.experimental import pallas as pl
from jax.experimental.pallas import tpu as pltpu
from jax.experimental.pallas import tpu_sc as plsc

N = 10000
E = 320000
D_IN = 128
H = 128
C = 40
K = 5

NC = 2
NS = 16
NW = NC * NS
BK = 80
NR = N // 2
ZB = 16
CR = 80


def _front(x, W, b1, src2, attr2):
    BM = 1000
    def body(x_ref, w_ref, b_ref, s_ref, a_ref,
             t_ref, r_ref, i8_ref, i4_ref, f_ref):
        o = lax.dot_general(
            x_ref[...], w_ref[...], (((1,), (0,)), ((), ())),
            precision=lax.Precision.HIGHEST,
            preferred_element_type=jnp.float32)
        t_ref[...] = o[:, :1024]
        r_ref[...] = o[:, 1024:] + b_ref[...]
        v = a_ref[...] * jnp.float32(K - 1)
        lo = jnp.floor(v)
        i0 = jnp.clip(lo.astype(jnp.int32), 0, K - 2)
        f_ref[...] = v - lo
        i8_ref[...] = s_ref[...] * 8 + i0
        i4_ref[...] = s_ref[...] * 4 + i0
    return pl.pallas_call(
        body,
        grid=(N // BM,),
        in_specs=[pl.BlockSpec((BM, 128), lambda i: (i, 0)),
                  pl.BlockSpec((128, 1152), lambda i: (0, 0)),
                  pl.BlockSpec((1, 128), lambda i: (0, 0)),
                  pl.BlockSpec((256, 128), lambda i: (i, 0)),
                  pl.BlockSpec((256, 128), lambda i: (i, 0))],
        out_specs=[pl.BlockSpec((BM, 1024), lambda i: (i, 0)),
                   pl.BlockSpec((BM, 128), lambda i: (i, 0)),
                   pl.BlockSpec((256, 128), lambda i: (i, 0)),
                   pl.BlockSpec((256, 128), lambda i: (i, 0)),
                   pl.BlockSpec((256, 128), lambda i: (i, 0))],
        out_shape=[jax.ShapeDtypeStruct((N, 1024), jnp.float32),
                   jax.ShapeDtypeStruct((N, 128), jnp.float32),
                   jax.ShapeDtypeStruct((2560, 128), jnp.int32),
                   jax.ShapeDtypeStruct((2560, 128), jnp.int32),
                   jax.ShapeDtypeStruct((2560, 128), jnp.float32)],
    )(x, W, b1, src2, attr2)


def _layer_mid(agg, c0, r1, W, brow):
    BM = 1000
    def body(a_ref, c0_ref, r1_ref, w_ref, b_ref, t_ref, r_ref, c_ref):
        rc = 1.0 / jnp.maximum(c0_ref[...], 1.0)
        h = jnp.maximum(a_ref[...] * rc + r1_ref[...], 0.0)
        o = lax.dot_general(
            h, w_ref[...], (((1,), (0,)), ((), ())),
            precision=lax.Precision.HIGHEST,
            preferred_element_type=jnp.float32) + b_ref[...]
        t_ref[...] = o[:, :512]
        r_ref[...] = o[:, 512:]
        c_ref[...] = jnp.broadcast_to(rc, (BM, H))
    return pl.pallas_call(
        body,
        grid=(N // BM,),
        in_specs=[pl.BlockSpec((BM, H), lambda i: (i, 0)),
                  pl.BlockSpec((BM, 1), lambda i: (i, 0)),
                  pl.BlockSpec((BM, H), lambda i: (i, 0)),
                  pl.BlockSpec((H, 576), lambda i: (0, 0)),
                  pl.BlockSpec((1, 576), lambda i: (0, 0))],
        out_specs=[pl.BlockSpec((BM, 512), lambda i: (i, 0)),
                   pl.BlockSpec((BM, 64), lambda i: (i, 0)),
                   pl.BlockSpec((BM, H), lambda i: (i, 0))],
        out_shape=[jax.ShapeDtypeStruct((N, 512), jnp.float32),
                   jax.ShapeDtypeStruct((N, 64), jnp.float32),
                   jax.ShapeDtypeStruct((N, H), jnp.float32)],
    )(agg, c0, r1, W, brow)


def _finalize(a0, a1, rcnt, r2):
    BM = 1000
    W64 = 64
    def body(a0_ref, a1_ref, c_ref, r2_ref, o_ref):
        z = (a0_ref[...] + a1_ref[...]) * c_ref[:, 0:1] + r2_ref[...]
        col = lax.broadcasted_iota(jnp.int32, (BM, W64), 1)
        valid = col < C
        m = jnp.max(jnp.where(valid, z, -jnp.inf), axis=1, keepdims=True)
        e = jnp.where(valid, jnp.exp(z - m), 0.0)
        s = jnp.sum(e, axis=1, keepdims=True)
        o_ref[...] = z - m - jnp.log(s)
    return pl.pallas_call(
        body,
        grid=(N // BM,),
        in_specs=[pl.BlockSpec((BM, W64), lambda i: (i, 0)),
                  pl.BlockSpec((BM, W64), lambda i: (i, 0)),
                  pl.BlockSpec((BM, H), lambda i: (i, 0)),
                  pl.BlockSpec((BM, W64), lambda i: (i, 0))],
        out_specs=pl.BlockSpec((BM, W64), lambda i: (i, 0)),
        out_shape=jax.ShapeDtypeStruct((N, W64), jnp.float32),
    )(a0, a1, rcnt, r2)


WB = 25
MR = 5008
NZ = MR // ZB


def _edge_pass(tab, idx3, dst3, frac2, feature_split):
    nwin = idx3.shape[1]
    arows = MR + CR if feature_split else MR
    out_types = [jax.ShapeDtypeStruct((NC, arows, 128), jnp.float32)]
    scratch = [
        pltpu.VMEM((WB, BK), jnp.int32),
        pltpu.VMEM((WB, BK), jnp.int32),
        pltpu.VMEM((WB, BK), jnp.float32),
        pltpu.VMEM((BK,), jnp.int32),
        pltpu.VMEM((BK,), jnp.int32),
        pltpu.VMEM((BK, 128), jnp.float32),
        pltpu.VMEM((BK, 128), jnp.float32),
        pltpu.VMEM((BK, 128), jnp.float32),
        pltpu.VMEM((BK, 128), jnp.float32),
        pltpu.VMEM((ZB, 128), jnp.float32),
        pltpu.VMEM_SHARED((arows, 128), jnp.float32),
        pltpu.SemaphoreType.DMA,
        pltpu.SemaphoreType.DMA,
        pltpu.SemaphoreType.DMA,
        pltpu.SemaphoreType.DMA,
    ]
    if feature_split:
        scratch += [
            pltpu.VMEM((CR, 128), jnp.float32),
            pltpu.VMEM((CR,), jnp.int32),
        ]
    mesh = plsc.VectorSubcoreMesh(core_axis_name="c", subcore_axis_name="s")

    @functools.partial(pl.kernel, out_type=out_types, mesh=mesh,
                       scratch_types=scratch)
    def kern(tab_h, idx_h, dst_h, frac_h, *refs):
        if feature_split:
            (out_h, idx_v, dst_v, frac_v, drow_a, drow_b, rows_a, rows_b,
             m_a, m_b, zero_v, agg_sh, semga, semgb, semsa, semsb,
             cnt_v, ident_v) = refs
        else:
            (out_h, idx_v, dst_v, frac_v, drow_a, drow_b, rows_a, rows_b,
             m_a, m_b, zero_v, agg_sh, semga, semgb, semsa, semsb) = refs
        cid = lax.axis_index("c")
        sid = lax.axis_index("s")
        gid = sid if feature_split else sid * NC + cid

        lane = lax.iota(jnp.int32, 16)
        zv = jnp.zeros((16,), jnp.float32)

        @pl.loop(0, ZB)
        def _zfill(i):
            for c_ in range(8):
                zero_v[i, pl.ds(c_ * 16, 16)] = zv

        if feature_split:
            @pl.loop(0, CR)
            def _cfill(i):
                for c_ in range(8):
                    cnt_v[i, pl.ds(c_ * 16, 16)] = zv
            for j in range(CR // 16):
                ident_v[pl.ds(j * 16, 16)] = lane + (MR + j * 16)
            @pl.when(sid == 0)
            def _czero():
                for j in range(CR // ZB):
                    pltpu.sync_copy(zero_v, agg_sh.at[pl.ds(MR + j * ZB, ZB)])

        nzch = (NZ - 1 - sid) // NS + 1
        @pl.loop(0, nzch)
        def _zinit(j):
            pltpu.sync_copy(zero_v, agg_sh.at[pl.ds((sid + j * NS) * ZB, ZB)])
        plsc.subcore_barrier()

        @pl.loop(0, nwin)
        def _win(w):
            pltpu.sync_copy(idx_h.at[gid, w], idx_v)
            pltpu.sync_copy(dst_h.at[gid, w], dst_v)
            pltpu.sync_copy(frac_h.at[gid, w], frac_v)
            if feature_split:
                off4 = cid * 4
                @pl.loop(0, WB)
                def _ixform(t):
                    for j in range(BK // 16):
                        idx_v[t, pl.ds(j * 16, 16)] = (
                            idx_v[t, pl.ds(j * 16, 16)] + off4)
            def compute(t, rows_v, m_v, drow_v):
                @pl.loop(0, BK // 16)
                def _grp(g):
                    fv = frac_v[t, pl.ds(g * 16, 16)]
                    dv = dst_v[t, pl.ds(g * 16, 16)]
                    drow_v[pl.ds(g * 16, 16)] = dv >> 1
                    for j in range(16):
                        b1 = jnp.full((16,), fv[j], jnp.float32)
                        b0 = 1.0 - b1
                        d = dv[j]
                        odd = (d & 1) == 1
                        i = g * 16 + j
                        for c_ in range(4):
                            g0 = rows_v[i, pl.ds(c_ * 16, 16)]
                            g1 = rows_v[i, pl.ds(64 + c_ * 16, 16)]
                            val = g0 * b0 + g1 * b1
                            m_v[i, pl.ds(c_ * 16, 16)] = jnp.where(
                                odd, zv, val)
                            m_v[i, pl.ds(64 + c_ * 16, 16)] = jnp.where(
                                odd, val, zv)
                        if feature_split:
                            r = d >> 7
                            cb = d & 112
                            oh = jnp.where(lane == (d & 15),
                                           jnp.float32(1.0), jnp.float32(0.0))
                            cnt_v[r, pl.ds(cb, 16)] = (
                                cnt_v[r, pl.ds(cb, 16)] + oh)

            pltpu.async_copy(tab_h.at[idx_v.at[0]], rows_a, semga)
            @pl.loop(0, WB // 2)
            def _pair(tt):
                t0 = 2 * tt
                pltpu.async_copy(tab_h.at[idx_v.at[t0 + 1]], rows_b, semgb)
                pltpu.make_async_copy(tab_h.at[idx_v.at[t0]], rows_a,
                                      semga).wait()
                @pl.when(tt > 0)
                def _wsa():
                    pltpu.make_async_copy(m_a, agg_sh.at[drow_a],
                                          semsa).wait()
                compute(t0, rows_a, m_a, drow_a)
                pltpu.async_copy(m_a, agg_sh.at[drow_a], semsa, add=True)
                @pl.when(tt + 1 < WB // 2)
                def _pre():
                    pltpu.async_copy(tab_h.at[idx_v.at[t0 + 2]], rows_a,
                                     semga)
                pltpu.make_async_copy(tab_h.at[idx_v.at[t0 + 1]], rows_b,
                                      semgb).wait()
                @pl.when(tt > 0)
                def _wsb():
                    pltpu.make_async_copy(m_b, agg_sh.at[drow_b],
                                          semsb).wait()
                compute(t0 + 1, rows_b, m_b, drow_b)
                pltpu.async_copy(m_b, agg_sh.at[drow_b], semsb, add=True)
            pltpu.make_async_copy(m_a, agg_sh.at[drow_a], semsa).wait()
            pltpu.make_async_copy(m_b, agg_sh.at[drow_b], semsb).wait()
            if WB % 2 == 1:
                t_last = WB - 1
                pltpu.async_copy(tab_h.at[idx_v.at[t_last]], rows_a,
                                 semga).wait()
                compute(t_last, rows_a, m_a, drow_a)
                pltpu.sync_copy(m_a, agg_sh.at[drow_a], add=True)

        if feature_split:
            pltpu.sync_copy(cnt_v, agg_sh.at[ident_v], add=True)
        plsc.subcore_barrier()

        @pl.loop(0, nzch)
        def _drain(j):
            off = (sid + j * NS) * ZB
            pltpu.sync_copy(agg_sh.at[pl.ds(off, ZB)],
                            out_h.at[cid, pl.ds(off, ZB)])
        if feature_split:
            @pl.when(sid == 0)
            def _cdrain():
                for j in range(CR // ZB):
                    pltpu.sync_copy(agg_sh.at[pl.ds(MR + j * ZB, ZB)],
                                    out_h.at[cid, pl.ds(MR + j * ZB, ZB)])

    return kern(tab, idx3, dst3, frac2)


def kernel(x, edge_index, edge_attr, W1, root1, bias1, W2, root2, bias2):
    f32 = jnp.float32
    blocks1 = []
    for c in range(2):
        for k in range(K - 1):
            blocks1.append(W1[k][:, c * 64:(c + 1) * 64])
            blocks1.append(W1[k + 1][:, c * 64:(c + 1) * 64])
    Wbig1 = jnp.concatenate(blocks1 + [root1], axis=1)
    b1row = bias1.reshape(1, 128)

    W2pad = jnp.pad(W2, ((0, 0), (0, 0), (0, 64 - C)))
    blocks2 = []
    for k in range(K - 1):
        blocks2.append(W2pad[k])
        blocks2.append(W2pad[k + 1])
    Wbig2 = jnp.concatenate(
        blocks2 + [root2, jnp.zeros((H, 64 - C), f32)], axis=1)
    b2row = jnp.zeros((1, 576), f32).at[0, 512:512 + C].set(bias2)

    src = edge_index[0]
    dst = edge_index[1]

    pad60 = ((0, 60), (0, 0))
    tabA, r1, idx8p, idx4p, fracp = _front(
        x, Wbig1, b1row, jnp.pad(src.reshape(2500, 128), pad60),
        jnp.pad(edge_attr[:, 0].reshape(2500, 128), pad60))
    tab1 = tabA.reshape(N * 8, 128)
    idx8, idx4, frac = idx8p[:2500], idx4p[:2500], fracp[:2500]
    nw1 = E // NS // (WB * BK)
    idx8g = idx8.reshape(NS, nw1, WB, BK)
    dst8g = dst.reshape(NS, nw1, WB, BK)
    frac8g = frac.reshape(NS, nw1, WB, BK)
    nw2 = E // NW // (WB * BK)
    idx4w = idx4.reshape(NW, nw2, WB, BK)
    dst4w = dst.reshape(NW, nw2, WB, BK)
    frac4w = frac.reshape(NW, nw2, WB, BK)
    (agg1p,) = _edge_pass(tab1, idx8g, dst8g, frac8g, feature_split=True)
    agg1 = jnp.concatenate(
        [agg1p[0][:MR].reshape(2 * MR, 64)[:N],
         agg1p[1][:MR].reshape(2 * MR, 64)[:N]], axis=1)
    c0 = agg1p[0][MR:].reshape(CR * 128, 1)[:N]

    t2, r2, rcnt = _layer_mid(agg1, c0, r1, Wbig2, b2row)
    tab2 = t2.reshape(N * 4, 128)

    (agg2p,) = _edge_pass(tab2, idx4w, dst4w, frac4w, feature_split=False)

    out64 = _finalize(agg2p[0].reshape(2 * MR, 64)[:N],
                      agg2p[1].reshape(2 * MR, 64)[:N], rcnt, r2)
    return out64[:, :C]
